# Initial kernel scaffold; baseline (speedup 1.0000x reference)
import jax
import jax.numpy as jnp
from jax.experimental import pallas as pl


def kernel(indices, embedding_matrix):
    raise NotImplementedError("write your pallas kernel here")


# speedup vs baseline: 1.0895x; 1.0895x over previous
import functools

import jax
import jax.numpy as jnp
from jax import lax
from jax.experimental import pallas as pl
from jax.experimental.pallas import tpu as pltpu
from jax.experimental.pallas import tpu_sc as plsc

_NUM_CORES = 2
_NUM_SUBCORES = 16
_NW = _NUM_CORES * _NUM_SUBCORES
_D = 32


@functools.partial(jax.jit, static_argnums=(2,))
def _gather(table, flat_idx, chunk):
    B = flat_idx.shape[0]
    assert B % _NW == 0
    b_per_w = B // _NW
    assert b_per_w % chunk == 0
    n_chunks = b_per_w // chunk

    mesh = plsc.VectorSubcoreMesh(
        core_axis_name="c",
        subcore_axis_name="s",
        num_cores=_NUM_CORES,
        num_subcores=_NUM_SUBCORES,
    )

    @functools.partial(
        pl.kernel,
        out_type=jax.ShapeDtypeStruct((B, _D), jnp.float32),
        mesh=mesh,
        compiler_params=pltpu.CompilerParams(use_tc_tiling_on_sc=False),
        scratch_types=[
            pltpu.VMEM((b_per_w,), jnp.int32),
            pltpu.VMEM((chunk, _D), jnp.float32),
            pltpu.SemaphoreType.DMA,
        ],
    )
    def k(tbl_hbm, idx_hbm, out_hbm, idx_v, rows_v, sem):
        wid = lax.axis_index("s") * _NUM_CORES + lax.axis_index("c")
        base = wid * b_per_w
        pltpu.sync_copy(idx_hbm.at[pl.ds(base, b_per_w)], idx_v)

        def body(i, carry):
            off = i * chunk
            pltpu.async_copy(
                tbl_hbm.at[idx_v.at[pl.ds(off, chunk)]], rows_v, sem
            ).wait()
            pltpu.sync_copy(rows_v, out_hbm.at[pl.ds(base + off, chunk)])
            return carry

        lax.fori_loop(0, n_chunks, body, 0)

    return k(table, flat_idx)


def kernel(indices, embedding_matrix):
    B0, S = indices.shape
    flat = indices.reshape(-1)
    out = _gather(embedding_matrix, flat, 1024)
    return out.reshape(B0, S, _D)


# speedup vs baseline: 1.3468x; 1.2361x over previous
import functools

import jax
import jax.numpy as jnp
from jax import lax
from jax.experimental import pallas as pl
from jax.experimental.pallas import tpu as pltpu
from jax.experimental.pallas import tpu_sc as plsc

_NUM_CORES = 2
_NUM_SUBCORES = 16
_NW = _NUM_CORES * _NUM_SUBCORES
_D = 32
_L = 16


@jax.jit
def _gather_native(table128, idx_t):
    S, B = idx_t.shape
    assert B % _NW == 0
    P = B // _NW

    mesh = plsc.VectorSubcoreMesh(
        core_axis_name="c",
        subcore_axis_name="s",
        num_cores=_NUM_CORES,
        num_subcores=_NUM_SUBCORES,
    )

    @functools.partial(
        pl.kernel,
        out_type=jax.ShapeDtypeStruct((S, _D, B), jnp.float32),
        mesh=mesh,
        compiler_params=pltpu.CompilerParams(
            use_tc_tiling_on_sc=True, needs_layout_passes=False
        ),
        scratch_types=[
            pltpu.VMEM((P,), jnp.int32),
            pltpu.VMEM((P,), jnp.int32),
            pltpu.VMEM((P,), jnp.int32),
            pltpu.VMEM((P, 128), jnp.float32),
            pltpu.VMEM((_D, P), jnp.float32),
            pltpu.SemaphoreType.DMA,
        ],
    )
    def k(tbl_hbm, idx_hbm, out_hbm, idx_v, idxq_v, colb_v, rows_v, obuf, sem):
        wid = lax.axis_index("s") * _NUM_CORES + lax.axis_index("c")
        b0 = wid * P
        lanes = lax.iota(jnp.int32, _L)

        def s_body(s, carry):
            pltpu.sync_copy(idx_hbm.at[s, pl.ds(b0, P)], idx_v)

            def prep(t, c):
                v = idx_v[pl.ds(t * _L, _L)]
                idxq_v[pl.ds(t * _L, _L)] = lax.shift_right_logical(v, 2)
                colb_v[pl.ds(t * _L, _L)] = lax.shift_left(v & 3, 5)
                return c

            lax.fori_loop(0, P // _L, prep, 0)

            pltpu.async_copy(tbl_hbm.at[idxq_v], rows_v, sem).wait()

            def blk_body(t, c):
                rows16 = t * _L + lanes
                colb16 = colb_v[pl.ds(t * _L, _L)]
                for d in range(_D):
                    vals = plsc.load_gather(rows_v, [rows16, colb16 + d])
                    obuf[d, pl.ds(t * _L, _L)] = vals
                return c

            lax.fori_loop(0, P // _L, blk_body, 0)

            pltpu.sync_copy(obuf, out_hbm.at[s, :, pl.ds(b0, P)])
            return carry

        lax.fori_loop(0, S, s_body, 0)

    return k(table128, idx_t)


def kernel(indices, embedding_matrix):
    B0, S = indices.shape
    tbl128 = embedding_matrix.reshape(-1, 128)
    idx_t = indices.T
    out = _gather_native(tbl128, idx_t)
    return jnp.transpose(out, (2, 0, 1))


# speedup vs baseline: 1.5707x; 1.1662x over previous
import functools

import jax
import jax.numpy as jnp
from jax import lax
from jax.experimental import pallas as pl
from jax.experimental.pallas import tpu as pltpu
from jax.experimental.pallas import tpu_sc as plsc

_NUM_CORES = 2
_NUM_SUBCORES = 16
_NW = _NUM_CORES * _NUM_SUBCORES
_D = 32
_L = 16
_C = 256


@jax.jit
def _gather_native(table128, idx_t):
    S, B = idx_t.shape
    assert B % _NW == 0
    P = B // _NW
    assert P % _C == 0
    halves = P // _C
    n_chunks = S * halves

    mesh = plsc.VectorSubcoreMesh(
        core_axis_name="c",
        subcore_axis_name="s",
        num_cores=_NUM_CORES,
        num_subcores=_NUM_SUBCORES,
    )

    @functools.partial(
        pl.kernel,
        out_type=jax.ShapeDtypeStruct((S, _D, B), jnp.float32),
        mesh=mesh,
        compiler_params=pltpu.CompilerParams(
            use_tc_tiling_on_sc=True, needs_layout_passes=False
        ),
        scratch_types=[
            pltpu.VMEM((S, P), jnp.int32),
            pltpu.VMEM((_C,), jnp.int32),
            pltpu.VMEM((_C,), jnp.int32),
            pltpu.VMEM((_C,), jnp.int32),
            pltpu.VMEM((_C,), jnp.int32),
            pltpu.VMEM((_C, 128), jnp.float32),
            pltpu.VMEM((_C, 128), jnp.float32),
            pltpu.VMEM((_D, _C), jnp.float32),
            pltpu.SemaphoreType.DMA,
            pltpu.SemaphoreType.DMA,
        ],
    )
    def k(tbl_hbm, idx_hbm, out_hbm, idx_all, idxq0, idxq1, colb0, colb1,
          rows0, rows1, obuf, sem0, sem1):
        wid = lax.axis_index("s") * _NUM_CORES + lax.axis_index("c")
        b0 = wid * P
        lanes = lax.iota(jnp.int32, _L)

        pltpu.sync_copy(idx_hbm.at[:, pl.ds(b0, P)], idx_all)

        def prep(i, idxq, colb):
            s = i // halves
            h = i % halves

            def body(t, c):
                v = idx_all[s, pl.ds(h * _C + t * _L, _L)]
                idxq[pl.ds(t * _L, _L)] = lax.shift_right_logical(v, 2)
                colb[pl.ds(t * _L, _L)] = lax.shift_left(v & 3, 5)
                return c

            lax.fori_loop(0, _C // _L, body, 0)

        def fire(idxq, rows, sem):
            pltpu.async_copy(tbl_hbm.at[idxq], rows, sem)

        def drain(idxq, rows, sem):
            pltpu.make_async_copy(tbl_hbm.at[idxq], rows, sem).wait()

        def consume(i, colb, rows):
            s = i // halves
            h = i % halves

            def body(t, c):
                rows16 = t * _L + lanes
                colb16 = colb[pl.ds(t * _L, _L)]
                for d in range(_D):
                    vals = plsc.load_gather(rows, [rows16, colb16 + d])
                    obuf[d, pl.ds(t * _L, _L)] = vals
                return c

            lax.fori_loop(0, _C // _L, body, 0)
            pltpu.sync_copy(
                obuf, out_hbm.at[s, :, pl.ds(b0 + h * _C, _C)]
            )

        prep(0, idxq0, colb0)
        fire(idxq0, rows0, sem0)

        def outer(g, carry):
            i0 = 2 * g

            @pl.when(i0 + 1 < n_chunks)
            def _():
                prep(i0 + 1, idxq1, colb1)
                fire(idxq1, rows1, sem1)

            drain(idxq0, rows0, sem0)
            consume(i0, colb0, rows0)

            @pl.when(i0 + 2 < n_chunks)
            def _():
                prep(i0 + 2, idxq0, colb0)
                fire(idxq0, rows0, sem0)

            @pl.when(i0 + 1 < n_chunks)
            def _():
                drain(idxq1, rows1, sem1)
                consume(i0 + 1, colb1, rows1)

            return carry

        lax.fori_loop(0, (n_chunks + 1) // 2, outer, 0)

    return k(table128, idx_t)


def kernel(indices, embedding_matrix):
    B0, S = indices.shape
    tbl128 = embedding_matrix.reshape(-1, 128)
    idx_t = indices.T
    out = _gather_native(tbl128, idx_t)
    return jnp.transpose(out, (2, 0, 1))
